# baseline (speedup 1.0000x reference)
import jax
import jax.numpy as jnp
import numpy as np
from jax.experimental import pallas as pl
from jax.experimental.pallas import tpu as pltpu

_ANCHORS = [1.3221, 1.73145, 3.19275, 4.00944, 5.05587,
            8.09892, 9.47112, 4.84053, 11.2364, 10.0071]
_A = 5
_G = 19
_HW = _G * _G
_N = _A * _HW
_NP = 1920
_RT = 128
_TILES = _NP // _RT
_NGT = 50
_NGTP = 64
_CONF = 0.5
_NMS_T = 0.45
_IOU_T = 0.5


def _sig(x):
    return 1.0 / (1.0 + jnp.exp(-x))


def _nms_body(chr_ref, chc_ref, tgt_ref, cr_ref, cc_ref, out_ref, s_scr):
    f32 = jnp.float32
    cr = cr_ref[...]
    ch = chr_ref[0]
    x_r = (_sig(ch[0:1]) + cr[0:1]) / 19.0
    y_r = (_sig(ch[1:2]) + cr[1:2]) / 19.0
    w_r = jnp.exp(ch[2:3]) * cr[2:3] / 19.0
    h_r = jnp.exp(ch[3:4]) * cr[3:4] / 19.0
    conf_r = _sig(ch[4:5])
    v_r = conf_r > _CONF
    l_r = x_r - w_r / 2.0
    r_r = x_r + w_r / 2.0
    t_r = y_r - h_r / 2.0
    b_r = y_r + h_r / 2.0
    area_r = w_r * h_r
    can_r = area_r * (_NMS_T / (1.0 + _NMS_T))
    cag_r = area_r * (_IOU_T / (1.0 + _IOU_T))

    cc = cc_ref[...]
    tc = chc_ref[0]
    x_c = (_sig(tc[:, 0:1]) + cc[:, 0:1]) / 19.0
    y_c = (_sig(tc[:, 1:2]) + cc[:, 1:2]) / 19.0
    w_c = jnp.exp(tc[:, 2:3]) * cc[:, 2:3] / 19.0
    h_c = jnp.exp(tc[:, 3:4]) * cc[:, 3:4] / 19.0
    conf_c = _sig(tc[:, 4:5])
    l_c = x_c - w_c / 2.0
    r_c = x_c + w_c / 2.0
    t_c = y_c - h_c / 2.0
    b_c = y_c + h_c / 2.0
    can_c = (w_c * h_c) * (_NMS_T / (1.0 + _NMS_T))

    def build_tile(off):
        sl = lambda v: v[off:off + _RT]
        l1, r1, t1, b1 = sl(l_c), sl(r_c), sl(t_c), sl(b_c)
        c1, ca1 = sl(conf_c), sl(can_c)
        cw = jnp.maximum(jnp.minimum(r1, r_r) - jnp.maximum(l1, l_r), 0.0)
        chh = jnp.maximum(jnp.minimum(b1, b_r) - jnp.maximum(t1, t_r), 0.0)
        overl = cw * chh > ca1 + can_r
        hi = off + _RT
        if off > 0:
            s = (c1 > conf_r[:, :off]) & overl[:, :off]
            s_scr[off:hi, 0:off] = s.astype(jnp.int8)
        ii = jax.lax.broadcasted_iota(jnp.int32, (_RT, 1), 0)
        jj = jax.lax.broadcasted_iota(jnp.int32, (1, _RT), 1)
        cm = conf_r[:, off:hi]
        rank = (c1 > cm) | ((c1 >= cm) & (ii < jj))
        s_scr[off:hi, off:hi] = (rank & overl[:, off:hi]).astype(jnp.int8)
        if hi < _NP:
            s = (c1 >= conf_r[:, hi:]) & overl[:, hi:]
            s_scr[off:hi, hi:] = s.astype(jnp.int8)

    for t in range(_TILES):
        build_tile(t * _RT)

    v_f = v_r.astype(f32)

    def step(k):
        sup = jax.lax.dot_general(
            k.astype(jnp.int8), s_scr[...],
            (((1,), (0,)), ((), ())), preferred_element_type=jnp.int32)
        return jnp.where(v_r & (sup < 1), 1.0, 0.0)

    def cond(c):
        old, new = c
        return jnp.any(old != new)

    def body(c):
        _, k = c
        return (k, step(k))

    _, keep = jax.lax.while_loop(
        cond, body, (jnp.full_like(v_f, -1.0), v_f))

    tg = tgt_ref[0]
    gx, gy, gw, gh = tg[:, 1:2], tg[:, 2:3], tg[:, 3:4], tg[:, 4:5]
    cw = jnp.maximum(
        jnp.minimum(gx + gw / 2.0, r_r) - jnp.maximum(gx - gw / 2.0, l_r), 0.0)
    chh = jnp.maximum(
        jnp.minimum(gy + gh / 2.0, b_r) - jnp.maximum(gy - gh / 2.0, t_r), 0.0)
    cag = (gw * gh) * (_IOU_T / (1.0 + _IOU_T))
    hit = (keep > 0.5) & (cw * chh > cag + cag_r)
    anyhit = jnp.max(hit.astype(f32), axis=1, keepdims=True)

    ind = (gx != 0).astype(f32)
    row_i = jax.lax.broadcasted_iota(jnp.int32, (_NGTP, _NGTP), 0)
    col_i = jax.lax.broadcasted_iota(jnp.int32, (_NGTP, _NGTP), 1)
    lower = (col_i <= row_i).astype(f32)
    counts = jax.lax.dot_general(
        lower, ind, (((1,), (0,)), ((), ())), preferred_element_type=f32)
    gnum = jax.lax.broadcasted_iota(jnp.int32, (_NGTP, 1), 0).astype(f32) + 1.0
    gvalid = counts == gnum

    t_sum = jnp.sum(gvalid.astype(f32))
    c_sum = jnp.sum((gvalid & (anyhit > 0.5)).astype(f32))
    p_sum = jnp.sum(keep)

    lane = jax.lax.broadcasted_iota(jnp.int32, (1, 128), 1)
    out_ref[0] = (jnp.where(lane == 0, t_sum, 0.0)
                  + jnp.where(lane == 1, p_sum, 0.0)
                  + jnp.where(lane == 2, c_sum, 0.0))


def kernel(output, target):
    f32 = jnp.float32
    B = output.shape[0]
    out5 = output.reshape(B, _A, 5 + 20, _HW)[:, :, :5, :]
    chr_ = out5.transpose(0, 2, 1, 3).reshape(B, 5, _N)
    chr_ = jnp.pad(chr_, ((0, 0), (0, 0), (0, _NP - _N)))
    chc_ = out5.transpose(0, 1, 3, 2).reshape(B, _N, 5)
    chc_ = jnp.pad(chc_, ((0, 0), (0, _NP - _N), (0, 3)))
    tgt = jnp.pad(target.reshape(B, _NGT, 5), ((0, 0), (0, _NGTP - _NGT), (0, 3)))

    hw = np.arange(_HW)
    cr = np.zeros((8, _NP), np.float32)
    cr[0, :_N] = np.tile(hw % _G, _A)
    cr[1, :_N] = np.tile(hw // _G, _A)
    cr[2, :_N] = np.repeat(np.asarray(_ANCHORS[0::2], np.float32), _HW)
    cr[3, :_N] = np.repeat(np.asarray(_ANCHORS[1::2], np.float32), _HW)
    cr[2:4, _N:] = 1.0
    cc = np.ascontiguousarray(cr.T)

    partial = pl.pallas_call(
        _nms_body,
        grid=(B,),
        in_specs=[
            pl.BlockSpec((1, 5, _NP), lambda i: (i, 0, 0)),
            pl.BlockSpec((1, _NP, 8), lambda i: (i, 0, 0)),
            pl.BlockSpec((1, _NGTP, 8), lambda i: (i, 0, 0)),
            pl.BlockSpec((8, _NP), lambda i: (0, 0)),
            pl.BlockSpec((_NP, 8), lambda i: (0, 0)),
        ],
        out_specs=pl.BlockSpec((1, 1, 128), lambda i: (i, 0, 0)),
        out_shape=jax.ShapeDtypeStruct((B, 1, 128), f32),
        scratch_shapes=[pltpu.VMEM((_NP, _NP), jnp.int8)],
    )(chr_, chc_, tgt, jnp.asarray(cr), jnp.asarray(cc))
    return jnp.sum(partial[:, 0, :3], axis=0)


# speedup vs baseline: 1.1768x; 1.1768x over previous
import jax
import jax.numpy as jnp
import numpy as np
from jax.experimental import pallas as pl
from jax.experimental.pallas import tpu as pltpu

_ANCHORS = [1.3221, 1.73145, 3.19275, 4.00944, 5.05587,
            8.09892, 9.47112, 4.84053, 11.2364, 10.0071]
_A = 5
_G = 19
_HW = _G * _G
_N = _A * _HW
_NP = 1920
_RT = 128
_TILES = _NP // _RT
_NGT = 50
_NGTP = 64
_CONF = 0.5
_NMS_T = 0.45
_IOU_T = 0.5


def _sig(x):
    return 1.0 / (1.0 + jnp.exp(-x))


def _nms_body(chr_ref, chc_ref, tgt_ref, cr_ref, cc_ref, out_ref, s_scr):
    f32 = jnp.float32
    cr = cr_ref[...]
    ch = chr_ref[0]
    x_r = (_sig(ch[0:1]) + cr[0:1]) / 19.0
    y_r = (_sig(ch[1:2]) + cr[1:2]) / 19.0
    w_r = jnp.exp(ch[2:3]) * cr[2:3] / 19.0
    h_r = jnp.exp(ch[3:4]) * cr[3:4] / 19.0
    conf_r = _sig(ch[4:5])
    v_r = conf_r > _CONF
    l_r = x_r - w_r / 2.0
    r_r = x_r + w_r / 2.0
    t_r = y_r - h_r / 2.0
    b_r = y_r + h_r / 2.0
    area_r = w_r * h_r
    can_r = area_r * (_NMS_T / (1.0 + _NMS_T))
    cag_r = area_r * (_IOU_T / (1.0 + _IOU_T))

    cc = cc_ref[...]
    tc = chc_ref[0]
    x_c = (_sig(tc[:, 0:1]) + cc[:, 0:1]) / 19.0
    y_c = (_sig(tc[:, 1:2]) + cc[:, 1:2]) / 19.0
    w_c = jnp.exp(tc[:, 2:3]) * cc[:, 2:3] / 19.0
    h_c = jnp.exp(tc[:, 3:4]) * cc[:, 3:4] / 19.0
    conf_c = _sig(tc[:, 4:5])
    l_c = x_c - w_c / 2.0
    r_c = x_c + w_c / 2.0
    t_c = y_c - h_c / 2.0
    b_c = y_c + h_c / 2.0
    can_c = (w_c * h_c) * (_NMS_T / (1.0 + _NMS_T))

    v_i8 = v_r.astype(jnp.int8)
    sup0 = jnp.zeros((1, _NP), jnp.int32)

    def build_tile(off, sup0):
        sl = lambda v: v[off:off + _RT]
        l1, r1, t1, b1 = sl(l_c), sl(r_c), sl(t_c), sl(b_c)
        c1, ca1 = sl(conf_c), sl(can_c)
        cw = jnp.maximum(jnp.minimum(r1, r_r) - jnp.maximum(l1, l_r), 0.0)
        chh = jnp.maximum(jnp.minimum(b1, b_r) - jnp.maximum(t1, t_r), 0.0)
        overl = cw * chh > ca1 + can_r
        hi = off + _RT
        parts = []
        if off > 0:
            parts.append(((c1 > conf_r[:, :off]) & overl[:, :off])
                         .astype(jnp.int8))
        ii = jax.lax.broadcasted_iota(jnp.int32, (_RT, 1), 0)
        jj = jax.lax.broadcasted_iota(jnp.int32, (1, _RT), 1)
        cm = conf_r[:, off:hi]
        rank = (c1 > cm) | ((c1 >= cm) & (ii < jj))
        parts.append((rank & overl[:, off:hi]).astype(jnp.int8))
        if hi < _NP:
            parts.append(((c1 >= conf_r[:, hi:]) & overl[:, hi:])
                         .astype(jnp.int8))
        s_i8 = jnp.concatenate(parts, axis=1)
        s_scr[off:hi, :] = s_i8
        return sup0 + jax.lax.dot_general(
            v_i8[:, off:hi], s_i8,
            (((1,), (0,)), ((), ())), preferred_element_type=jnp.int32)

    for t in range(_TILES):
        sup0 = build_tile(t * _RT, sup0)

    v_f = v_r.astype(f32)

    def step(k):
        sup = jax.lax.dot_general(
            k.astype(jnp.int8), s_scr[...],
            (((1,), (0,)), ((), ())), preferred_element_type=jnp.int32)
        return jnp.where(v_r & (sup < 1), 1.0, 0.0)

    def cond(c):
        old, new = c
        return jnp.any(old != new)

    def body(c):
        _, k = c
        return (k, step(k))

    k1 = jnp.where(v_r & (sup0 < 1), 1.0, 0.0)
    _, keep = jax.lax.while_loop(cond, body, (v_f, k1))

    tg = tgt_ref[0]
    gx, gy, gw, gh = tg[:, 1:2], tg[:, 2:3], tg[:, 3:4], tg[:, 4:5]
    cw = jnp.maximum(
        jnp.minimum(gx + gw / 2.0, r_r) - jnp.maximum(gx - gw / 2.0, l_r), 0.0)
    chh = jnp.maximum(
        jnp.minimum(gy + gh / 2.0, b_r) - jnp.maximum(gy - gh / 2.0, t_r), 0.0)
    cag = (gw * gh) * (_IOU_T / (1.0 + _IOU_T))
    hit = (keep > 0.5) & (cw * chh > cag + cag_r)
    anyhit = jnp.max(hit.astype(f32), axis=1, keepdims=True)

    ind = (gx != 0).astype(f32)
    row_i = jax.lax.broadcasted_iota(jnp.int32, (_NGTP, _NGTP), 0)
    col_i = jax.lax.broadcasted_iota(jnp.int32, (_NGTP, _NGTP), 1)
    lower = (col_i <= row_i).astype(f32)
    counts = jax.lax.dot_general(
        lower, ind, (((1,), (0,)), ((), ())), preferred_element_type=f32)
    gnum = jax.lax.broadcasted_iota(jnp.int32, (_NGTP, 1), 0).astype(f32) + 1.0
    gvalid = counts == gnum

    t_sum = jnp.sum(gvalid.astype(f32))
    c_sum = jnp.sum((gvalid & (anyhit > 0.5)).astype(f32))
    p_sum = jnp.sum(keep)

    lane = jax.lax.broadcasted_iota(jnp.int32, (1, 128), 1)
    out_ref[0] = (jnp.where(lane == 0, t_sum, 0.0)
                  + jnp.where(lane == 1, p_sum, 0.0)
                  + jnp.where(lane == 2, c_sum, 0.0))


def kernel(output, target):
    f32 = jnp.float32
    B = output.shape[0]
    out5 = output.reshape(B, _A, 5 + 20, _HW)[:, :, :5, :]
    chr_ = out5.transpose(0, 2, 1, 3).reshape(B, 5, _N)
    chr_ = jnp.pad(chr_, ((0, 0), (0, 0), (0, _NP - _N)))
    chc_ = out5.transpose(0, 1, 3, 2).reshape(B, _N, 5)
    chc_ = jnp.pad(chc_, ((0, 0), (0, _NP - _N), (0, 3)))
    tgt = jnp.pad(target.reshape(B, _NGT, 5), ((0, 0), (0, _NGTP - _NGT), (0, 3)))

    hw = np.arange(_HW)
    cr = np.zeros((8, _NP), np.float32)
    cr[0, :_N] = np.tile(hw % _G, _A)
    cr[1, :_N] = np.tile(hw // _G, _A)
    cr[2, :_N] = np.repeat(np.asarray(_ANCHORS[0::2], np.float32), _HW)
    cr[3, :_N] = np.repeat(np.asarray(_ANCHORS[1::2], np.float32), _HW)
    cr[2:4, _N:] = 1.0
    cc = np.ascontiguousarray(cr.T)

    partial = pl.pallas_call(
        _nms_body,
        grid=(B,),
        in_specs=[
            pl.BlockSpec((1, 5, _NP), lambda i: (i, 0, 0)),
            pl.BlockSpec((1, _NP, 8), lambda i: (i, 0, 0)),
            pl.BlockSpec((1, _NGTP, 8), lambda i: (i, 0, 0)),
            pl.BlockSpec((8, _NP), lambda i: (0, 0)),
            pl.BlockSpec((_NP, 8), lambda i: (0, 0)),
        ],
        out_specs=pl.BlockSpec((1, 1, 128), lambda i: (i, 0, 0)),
        out_shape=jax.ShapeDtypeStruct((B, 1, 128), f32),
        scratch_shapes=[pltpu.VMEM((_NP, _NP), jnp.int8)],
    )(chr_, chc_, tgt, jnp.asarray(cr), jnp.asarray(cc))
    return jnp.sum(partial[:, 0, :3], axis=0)


# speedup vs baseline: 1.1768x; 1.0000x over previous
import jax
import jax.numpy as jnp
import numpy as np
from jax.experimental import pallas as pl
from jax.experimental.pallas import tpu as pltpu

_ANCHORS = [1.3221, 1.73145, 3.19275, 4.00944, 5.05587,
            8.09892, 9.47112, 4.84053, 11.2364, 10.0071]
_A = 5
_G = 19
_HW = _G * _G
_N = _A * _HW
_NP = 1920
_RT = 128
_TILES = _NP // _RT
_NGT = 50
_NGTP = 64
_CONF = 0.5
_NMS_T = 0.45
_IOU_T = 0.5


def _sig(x):
    return 1.0 / (1.0 + jnp.exp(-x))


def _nms_body(chr_ref, chc_ref, tgt_ref, cr_ref, cc_ref, out_ref):
    f32 = jnp.float32
    cr = cr_ref[...]
    ch = chr_ref[0]
    x_r = (_sig(ch[0:1]) + cr[0:1]) / 19.0
    y_r = (_sig(ch[1:2]) + cr[1:2]) / 19.0
    w_r = jnp.exp(ch[2:3]) * cr[2:3] / 19.0
    h_r = jnp.exp(ch[3:4]) * cr[3:4] / 19.0
    conf_r = _sig(ch[4:5])
    v_r = conf_r > _CONF
    l_r = x_r - w_r / 2.0
    r_r = x_r + w_r / 2.0
    t_r = y_r - h_r / 2.0
    b_r = y_r + h_r / 2.0
    area_r = w_r * h_r
    can_r = area_r * (_NMS_T / (1.0 + _NMS_T))
    cag_r = area_r * (_IOU_T / (1.0 + _IOU_T))

    cc = cc_ref[...]
    tc = chc_ref[0]
    x_c = (_sig(tc[:, 0:1]) + cc[:, 0:1]) / 19.0
    y_c = (_sig(tc[:, 1:2]) + cc[:, 1:2]) / 19.0
    w_c = jnp.exp(tc[:, 2:3]) * cc[:, 2:3] / 19.0
    h_c = jnp.exp(tc[:, 3:4]) * cc[:, 3:4] / 19.0
    conf_c = _sig(tc[:, 4:5])
    l_c = x_c - w_c / 2.0
    r_c = x_c + w_c / 2.0
    t_c = y_c - h_c / 2.0
    b_c = y_c + h_c / 2.0
    can_c = (w_c * h_c) * (_NMS_T / (1.0 + _NMS_T))

    v_i8 = v_r.astype(jnp.int8)
    sup0 = jnp.zeros((1, _NP), jnp.int32)

    def build_tile(off, sup0):
        sl = lambda v: v[off:off + _RT]
        l1, r1, t1, b1 = sl(l_c), sl(r_c), sl(t_c), sl(b_c)
        c1, ca1 = sl(conf_c), sl(can_c)
        cw = jnp.maximum(jnp.minimum(r1, r_r) - jnp.maximum(l1, l_r), 0.0)
        chh = jnp.maximum(jnp.minimum(b1, b_r) - jnp.maximum(t1, t_r), 0.0)
        overl = cw * chh > ca1 + can_r
        hi = off + _RT
        parts = []
        if off > 0:
            parts.append(((c1 > conf_r[:, :off]) & overl[:, :off])
                         .astype(jnp.int8))
        ii = jax.lax.broadcasted_iota(jnp.int32, (_RT, 1), 0)
        jj = jax.lax.broadcasted_iota(jnp.int32, (1, _RT), 1)
        cm = conf_r[:, off:hi]
        rank = (c1 > cm) | ((c1 >= cm) & (ii < jj))
        parts.append((rank & overl[:, off:hi]).astype(jnp.int8))
        if hi < _NP:
            parts.append(((c1 >= conf_r[:, hi:]) & overl[:, hi:])
                         .astype(jnp.int8))
        s_i8 = jnp.concatenate(parts, axis=1)
        d = jax.lax.dot_general(
            v_i8[:, off:hi], s_i8,
            (((1,), (0,)), ((), ())), preferred_element_type=jnp.int32)
        return s_i8, sup0 + d

    tiles = []
    for t in range(_TILES):
        s_i8, sup0 = build_tile(t * _RT, sup0)
        tiles.append(s_i8)
    s_all = jnp.concatenate(tiles, axis=0)

    v_f = v_r.astype(f32)

    def step(k):
        sup = jax.lax.dot_general(
            k.astype(jnp.int8), s_all,
            (((1,), (0,)), ((), ())), preferred_element_type=jnp.int32)
        return jnp.where(v_r & (sup < 1), 1.0, 0.0)

    def cond(c):
        old, new = c
        return jnp.any(old != new)

    def body(c):
        _, k = c
        return (k, step(k))

    k1 = jnp.where(v_r & (sup0 < 1), 1.0, 0.0)
    _, keep = jax.lax.while_loop(cond, body, (v_f, k1))

    tg = tgt_ref[0]
    gx, gy, gw, gh = tg[:, 1:2], tg[:, 2:3], tg[:, 3:4], tg[:, 4:5]
    cw = jnp.maximum(
        jnp.minimum(gx + gw / 2.0, r_r) - jnp.maximum(gx - gw / 2.0, l_r), 0.0)
    chh = jnp.maximum(
        jnp.minimum(gy + gh / 2.0, b_r) - jnp.maximum(gy - gh / 2.0, t_r), 0.0)
    cag = (gw * gh) * (_IOU_T / (1.0 + _IOU_T))
    hit = (keep > 0.5) & (cw * chh > cag + cag_r)
    anyhit = jnp.max(hit.astype(f32), axis=1, keepdims=True)

    ind = (gx != 0).astype(f32)
    row_i = jax.lax.broadcasted_iota(jnp.int32, (_NGTP, _NGTP), 0)
    col_i = jax.lax.broadcasted_iota(jnp.int32, (_NGTP, _NGTP), 1)
    lower = (col_i <= row_i).astype(f32)
    counts = jax.lax.dot_general(
        lower, ind, (((1,), (0,)), ((), ())), preferred_element_type=f32)
    gnum = jax.lax.broadcasted_iota(jnp.int32, (_NGTP, 1), 0).astype(f32) + 1.0
    gvalid = counts == gnum

    t_sum = jnp.sum(gvalid.astype(f32))
    c_sum = jnp.sum((gvalid & (anyhit > 0.5)).astype(f32))
    p_sum = jnp.sum(keep)

    lane = jax.lax.broadcasted_iota(jnp.int32, (1, 128), 1)
    out_ref[0] = (jnp.where(lane == 0, t_sum, 0.0)
                  + jnp.where(lane == 1, p_sum, 0.0)
                  + jnp.where(lane == 2, c_sum, 0.0))


def kernel(output, target):
    f32 = jnp.float32
    B = output.shape[0]
    out5 = output.reshape(B, _A, 5 + 20, _HW)[:, :, :5, :]
    chr_ = out5.transpose(0, 2, 1, 3).reshape(B, 5, _N)
    chr_ = jnp.pad(chr_, ((0, 0), (0, 0), (0, _NP - _N)))
    chc_ = out5.transpose(0, 1, 3, 2).reshape(B, _N, 5)
    chc_ = jnp.pad(chc_, ((0, 0), (0, _NP - _N), (0, 3)))
    tgt = jnp.pad(target.reshape(B, _NGT, 5), ((0, 0), (0, _NGTP - _NGT), (0, 3)))

    hw = np.arange(_HW)
    cr = np.zeros((8, _NP), np.float32)
    cr[0, :_N] = np.tile(hw % _G, _A)
    cr[1, :_N] = np.tile(hw // _G, _A)
    cr[2, :_N] = np.repeat(np.asarray(_ANCHORS[0::2], np.float32), _HW)
    cr[3, :_N] = np.repeat(np.asarray(_ANCHORS[1::2], np.float32), _HW)
    cr[2:4, _N:] = 1.0
    cc = np.ascontiguousarray(cr.T)

    partial = pl.pallas_call(
        _nms_body,
        grid=(B,),
        in_specs=[
            pl.BlockSpec((1, 5, _NP), lambda i: (i, 0, 0)),
            pl.BlockSpec((1, _NP, 8), lambda i: (i, 0, 0)),
            pl.BlockSpec((1, _NGTP, 8), lambda i: (i, 0, 0)),
            pl.BlockSpec((8, _NP), lambda i: (0, 0)),
            pl.BlockSpec((_NP, 8), lambda i: (0, 0)),
        ],
        out_specs=pl.BlockSpec((1, 1, 128), lambda i: (i, 0, 0)),
        out_shape=jax.ShapeDtypeStruct((B, 1, 128), f32),
    )(chr_, chc_, tgt, jnp.asarray(cr), jnp.asarray(cc))
    return jnp.sum(partial[:, 0, :3], axis=0)
